# baseline (device time: 6475 ns/iter reference)
import jax
import jax.numpy as jnp
from jax import lax
from jax.experimental import pallas as pl
from jax.experimental.pallas import tpu as pltpu

N_DEV = 8


def kernel(x):
    m_per, n = x.shape

    def body(x_ref, out_ref):
        my = lax.axis_index("i")
        barrier_sem = pltpu.get_barrier_semaphore()
        for k in range(1, N_DEV):
            tgt = lax.rem(my + k, N_DEV)
            pl.semaphore_signal(
                barrier_sem, inc=1,
                device_id=(tgt,), device_id_type=pl.DeviceIdType.MESH,
            )
        out_ref[:, :] = jnp.sum(x_ref[:], axis=0, keepdims=True)
        pl.semaphore_wait(barrier_sem, N_DEV - 1)

    return pl.pallas_call(
        body,
        out_shape=jax.ShapeDtypeStruct((1, n), jnp.float32),
        in_specs=[pl.BlockSpec(memory_space=pltpu.VMEM)],
        out_specs=pl.BlockSpec(memory_space=pltpu.VMEM),
        compiler_params=pltpu.CompilerParams(collective_id=0),
    )(x)


# device time: 5655 ns/iter; 1.1450x vs baseline; 1.1450x over previous
import jax
import jax.numpy as jnp
from jax import lax
from jax.experimental import pallas as pl
from jax.experimental.pallas import tpu as pltpu

N_DEV = 8


def kernel(x):
    m_per, n = x.shape

    def body(x_ref, out_ref):
        my = lax.axis_index("i")
        barrier_sem = pltpu.get_barrier_semaphore()
        tgt = lax.rem(my + 1, N_DEV)
        pl.semaphore_signal(
            barrier_sem, inc=1,
            device_id=(tgt,), device_id_type=pl.DeviceIdType.MESH,
        )
        out_ref[:, :] = jnp.sum(x_ref[:], axis=0, keepdims=True)
        pl.semaphore_wait(barrier_sem, 1)

    return pl.pallas_call(
        body,
        out_shape=jax.ShapeDtypeStruct((1, n), jnp.float32),
        in_specs=[pl.BlockSpec(memory_space=pltpu.VMEM)],
        out_specs=pl.BlockSpec(memory_space=pltpu.VMEM),
        compiler_params=pltpu.CompilerParams(collective_id=0),
    )(x)
